# baseline (speedup 1.0000x reference)
import functools

import jax
import jax.numpy as jnp
from jax import lax
from jax.experimental import pallas as pl
from jax.experimental.pallas import tpu as pltpu

B = 128
N = 32768
KCOL = 1000
BN = 2048
NB = N // BN
NEG = float("-inf")


def _fold(data):
    m = data[:, 0:128]
    for j in range(1, BN // 128):
        m = jnp.maximum(m, data[:, j * 128:(j + 1) * 128])
    return m


def _body(in_ref, out_ref, acc_ref):
    i = pl.program_id(0)

    @pl.when(i == 0)
    def _():
        acc_ref[...] = jnp.full((B, 128), NEG, jnp.float32)

    data = in_ref[...]

    @pl.when(i < NB - 1)
    def _():
        acc_ref[...] = jnp.maximum(acc_ref[...], _fold(data))
        out_ref[...] = jnp.zeros((B, BN), jnp.float32)

    @pl.when(i == NB - 1)
    def _():
        col = lax.broadcasted_iota(jnp.int32, (B, BN), 1)
        neg = jnp.float32(NEG)
        m_b = jnp.max(jnp.where(col < KCOL, data, neg), axis=1)
        m_a0 = jnp.max(jnp.where(col > KCOL, data, neg), axis=1)
        v = data[:, KCOL]
        m_a = jnp.maximum(m_a0, jnp.max(acc_ref[...], axis=1))
        flag = jnp.where(jnp.logical_and(v > m_b, v >= m_a),
                         jnp.float32(1.0), jnp.float32(0.0))
        out_ref[...] = jnp.where(col == KCOL, flag[:, None],
                                 jnp.float32(0.0))


@jax.jit
def _run(inputs):
    def idx(i):
        return (0, (i + 1) % NB)

    return pl.pallas_call(
        _body,
        grid=(NB,),
        in_specs=[pl.BlockSpec((B, BN), idx)],
        out_specs=pl.BlockSpec((B, BN), idx),
        out_shape=jax.ShapeDtypeStruct((B, N), jnp.float32),
        scratch_shapes=[pltpu.VMEM((B, 128), jnp.float32)],
    )(inputs)


def kernel(inputs):
    return _run(inputs)


# speedup vs baseline: 1.3983x; 1.3983x over previous
import functools

import jax
import jax.numpy as jnp
from jax import lax
from jax.experimental import pallas as pl
from jax.experimental.pallas import tpu as pltpu

B = 128
N = 32768
KCOL = 1000
BN = 8192
NB = N // BN
NEG = float("-inf")


def _fold(data):
    m = data[:, 0:128]
    for j in range(1, BN // 128):
        m = jnp.maximum(m, data[:, j * 128:(j + 1) * 128])
    return m


def _body(in_ref, out_ref, acc_ref):
    i = pl.program_id(0)

    @pl.when(i == 0)
    def _():
        acc_ref[...] = jnp.full((B, 128), NEG, jnp.float32)

    data = in_ref[...]

    @pl.when(i < NB - 1)
    def _():
        acc_ref[...] = jnp.maximum(acc_ref[...], _fold(data))
        out_ref[...] = jnp.zeros((B, BN), jnp.float32)

    @pl.when(i == NB - 1)
    def _():
        col = lax.broadcasted_iota(jnp.int32, (B, BN), 1)
        neg = jnp.float32(NEG)
        m_b = jnp.max(jnp.where(col < KCOL, data, neg), axis=1)
        m_a0 = jnp.max(jnp.where(col > KCOL, data, neg), axis=1)
        v = data[:, KCOL]
        m_a = jnp.maximum(m_a0, jnp.max(acc_ref[...], axis=1))
        flag = jnp.where(jnp.logical_and(v > m_b, v >= m_a),
                         jnp.float32(1.0), jnp.float32(0.0))
        out_ref[...] = jnp.where(col == KCOL, flag[:, None],
                                 jnp.float32(0.0))


@jax.jit
def _run(inputs):
    def idx(i):
        return (0, (i + 1) % NB)

    return pl.pallas_call(
        _body,
        grid=(NB,),
        in_specs=[pl.BlockSpec((B, BN), idx)],
        out_specs=pl.BlockSpec((B, BN), idx),
        out_shape=jax.ShapeDtypeStruct((B, N), jnp.float32),
        scratch_shapes=[pltpu.VMEM((B, 128), jnp.float32)],
    )(inputs)


def kernel(inputs):
    return _run(inputs)


# speedup vs baseline: 1.6840x; 1.2043x over previous
import functools

import jax
import jax.numpy as jnp
from jax import lax
from jax.experimental import pallas as pl
from jax.experimental.pallas import tpu as pltpu

B = 128
N = 32768
KCOL = 1000
BN = 16384
NB = N // BN
NEG = float("-inf")


def _fold(data):
    m = data[:, 0:128]
    for j in range(1, BN // 128):
        m = jnp.maximum(m, data[:, j * 128:(j + 1) * 128])
    return m


def _body(in_ref, out_ref, acc_ref):
    i = pl.program_id(0)

    @pl.when(i == 0)
    def _():
        acc_ref[...] = jnp.full((B, 128), NEG, jnp.float32)

    data = in_ref[...]

    @pl.when(i < NB - 1)
    def _():
        acc_ref[...] = jnp.maximum(acc_ref[...], _fold(data))
        out_ref[...] = jnp.zeros((B, BN), jnp.float32)

    @pl.when(i == NB - 1)
    def _():
        col = lax.broadcasted_iota(jnp.int32, (B, BN), 1)
        neg = jnp.float32(NEG)
        m_b = jnp.max(jnp.where(col < KCOL, data, neg), axis=1)
        m_a0 = jnp.max(jnp.where(col > KCOL, data, neg), axis=1)
        v = data[:, KCOL]
        m_a = jnp.maximum(m_a0, jnp.max(acc_ref[...], axis=1))
        flag = jnp.where(jnp.logical_and(v > m_b, v >= m_a),
                         jnp.float32(1.0), jnp.float32(0.0))
        out_ref[...] = jnp.where(col == KCOL, flag[:, None],
                                 jnp.float32(0.0))


@jax.jit
def _run(inputs):
    def idx(i):
        return (0, (i + 1) % NB)

    return pl.pallas_call(
        _body,
        grid=(NB,),
        in_specs=[pl.BlockSpec((B, BN), idx)],
        out_specs=pl.BlockSpec((B, BN), idx),
        out_shape=jax.ShapeDtypeStruct((B, N), jnp.float32),
        scratch_shapes=[pltpu.VMEM((B, 128), jnp.float32)],
    )(inputs)


def kernel(inputs):
    return _run(inputs)
